# baseline (speedup 1.0000x reference)
import functools

import jax
import jax.numpy as jnp
from jax import lax
from jax.experimental import pallas as pl
from jax.experimental.pallas import tpu as pltpu
from jax.experimental.pallas import tpu_sc as plsc

BATCH = 16384
FIELDS = 26
D = 64
W = 128
B = BATCH * FIELDS
PAIRS = 500000
NW = 32
BPW = B // NW
CH = 128
NCH = BPW // CH


def _build():
    mesh = plsc.VectorSubcoreMesh(core_axis_name="c", subcore_axis_name="s")

    @functools.partial(
        pl.kernel,
        mesh=mesh,
        out_type=jax.ShapeDtypeStruct((B // 2, W), jnp.float32),
        scratch_types=[
            pltpu.VMEM((NCH, CH), jnp.int32),
            pltpu.VMEM((NCH, CH), jnp.int32),
            pltpu.VMEM((2, CH, W), jnp.float32),
            pltpu.VMEM((CH // 2, W), jnp.float32),
            pltpu.SemaphoreType.DMA,
            pltpu.SemaphoreType.DMA,
        ],
        compiler_params=pltpu.CompilerParams(use_tc_tiling_on_sc=True),
    )
    def emb_kernel(idx_hbm, table_hbm, out_hbm, idx_v, pidx_v, buf_v, ex_v,
                   g0, g1):
        gsems = (g0, g1)
        wid = lax.axis_index("s") * 2 + lax.axis_index("c")
        obase = wid * (BPW // 2)

        pltpu.sync_copy(idx_hbm.at[wid], idx_v)

        def precompute(j, carry):
            for k in range(CH // 16):
                t16 = idx_v[j, pl.ds(16 * k, 16)]
                pidx_v[j, pl.ds(16 * k, 16)] = lax.shift_right_logical(t16, 1)
            return carry

        lax.fori_loop(0, NCH, precompute, 0)

        def gather(j, bb):
            pltpu.async_copy(table_hbm.at[pidx_v.at[j]], buf_v.at[bb], gsems[bb])

        def gather_wait(bb):
            pltpu.make_async_copy(
                table_hbm.at[pidx_v.at[0]], buf_v.at[bb], gsems[bb]
            ).wait()

        def extract(j, bb):
            def body(k, carry):
                t16 = idx_v[j, pl.ds(16 * k, 16)]
                i0 = 16 * k
                r0 = 8 * k
                for ii in range(16):
                    t = t16[ii]
                    s = lax.shift_left(lax.bitwise_and(t, 1), 6)
                    d = (ii & 1) * 64
                    for m in range(4):
                        v = buf_v[bb, i0 + ii, pl.ds(s + 16 * m, 16)]
                        ex_v[r0 + (ii >> 1), pl.ds(d + 16 * m, 16)] = v
                return carry

            lax.fori_loop(0, CH // 16, body, 0)

        gather(0, 0)
        gather(1, 1)

        def group(g, carry):
            for b in (0, 1):
                j = 2 * g + b
                gather_wait(b)
                extract(j, b)
                nxt = jnp.minimum(j + 2, NCH - 1)
                gather(nxt, b)
                pltpu.sync_copy(
                    ex_v, out_hbm.at[pl.ds(obase + j * (CH // 2), CH // 2)]
                )
            return carry

        lax.fori_loop(0, NCH // 2, group, 0)
        gather_wait(0)
        gather_wait(1)

    return emb_kernel


_emb = _build()


@jax.jit
def kernel(token_ids, weight):
    pairs = weight.reshape(PAIRS, W)
    idx = token_ids.reshape(NW, NCH, CH).astype(jnp.int32)
    out = _emb(idx, pairs)
    return out.reshape(BATCH, FIELDS, D)
